# baseline (speedup 1.0000x reference)
import functools

import jax
import jax.numpy as jnp
from jax import lax
from jax.experimental import pallas as pl
from jax.experimental.pallas import tpu as pltpu
from jax.experimental.pallas import tpu_sc as plsc

M = 8192
K = 20
C = 64
CP = 128
BQ = 256
BQ3 = 512
E = M * K


_NT = M // 128
_R = 5


def _knn_body(pT_ref, p8_ref, f_ref, wd_ref, wg_ref, b1_ref,
              idx_ref, a_ref, g_ref, s_ref, wv_ref, wi_ref):
    pT = pT_ref[...]
    sq = jnp.sum(pT * pT, axis=0, keepdims=True)
    pb = p8_ref[...]
    s_ref[...] = sq + jax.lax.dot(
        pb, pT, preferred_element_type=jnp.float32)

    f = f_ref[...]
    a_ref[...] = jax.lax.dot(
        f, wd_ref[...], preferred_element_type=jnp.float32) + b1_ref[...]
    g_ref[...] = jax.lax.dot(
        f, wg_ref[...], preferred_element_type=jnp.float32)

    INF = jnp.float32(jnp.inf)
    lane128 = lax.broadcasted_iota(jnp.int32, (BQ, 128), 1)
    lane32 = lax.broadcasted_iota(jnp.int32, (BQ, 32), 1)

    def _tree(vals, op):
        while len(vals) > 1:
            vals = [op(vals[i], vals[i + 1]) if i + 1 < len(vals) else vals[i]
                    for i in range(0, len(vals), 2)]
        return vals[0]

    def round_body(r, ti_prev):
        segs = []
        for t in range(_NT):
            seg = s_ref[:, t * 128:(t + 1) * 128]
            segm = jnp.where(ti_prev == t, INF, seg)
            s_ref[:, t * 128:(t + 1) * 128] = segm
            segs.append(segm)
        cm = _tree(segs, jnp.minimum)
        arr = s_ref[...]
        packed = [jnp.where(arr[:, t * 128:(t + 1) * 128] == cm, t, _NT)
                  for t in range(_NT)]
        ti = _tree(packed, jnp.minimum)
        wv_ref[r] = cm
        wi_ref[r] = ti * 128 + lane128
        return ti

    lax.fori_loop(0, _R, round_body, jnp.full((BQ, 128), _NT, jnp.int32))

    BIG = jnp.int32(M)

    def merge_body(k, carry):
        h, hi, ptr, acc = carry
        m = jnp.min(h, axis=1, keepdims=True)
        hit = h == m
        gm = jnp.min(jnp.where(hit, hi, BIG), axis=1, keepdims=True)
        win = hit & (hi == gm)
        ptr = ptr + win.astype(jnp.int32)
        hn = jnp.full((BQ, 128), INF, jnp.float32)
        hin = jnp.full((BQ, 128), BIG, jnp.int32)
        for r in range(_R):
            cond = ptr == r
            hn = jnp.where(cond, wv_ref[r], hn)
            hin = jnp.where(cond, wi_ref[r], hin)
        return hn, hin, ptr, jnp.where(lane32 == k, gm, acc)

    _, _, ptr, acc = lax.fori_loop(
        0, K, merge_body,
        (wv_ref[0], wi_ref[0], jnp.zeros((BQ, 128), jnp.int32),
         jnp.zeros((BQ, 32), jnp.int32)))
    idx_ref[...] = acc[:, :K]

    viol = jnp.max(ptr) >= _R

    def fallback():
        s_ref[...] = sq + jax.lax.dot(
            pb, pT, preferred_element_type=jnp.float32)
        iota = lax.broadcasted_iota(jnp.int32, (BQ, M), 1)

        def body(k, acc):
            arr = s_ref[...]
            m = jnp.min(arr, axis=1, keepdims=True)
            hit = arr == m
            idxv = jnp.min(jnp.where(hit, iota, M), axis=1, keepdims=True)
            s_ref[...] = jnp.where(iota == idxv, INF, arr)
            return jnp.where(lane32 == k, idxv, acc)

        acc = lax.fori_loop(0, K, body, jnp.zeros((BQ, 32), jnp.int32))
        idx_ref[...] = acc[:, :K]

    lax.cond(viol, fallback, lambda: None)


def _knn_call(pT, p8, f, wd, wg, b1):
    return pl.pallas_call(
        _knn_body,
        grid=(M // BQ,),
        in_specs=[
            pl.BlockSpec((8, M), lambda i: (0, 0)),
            pl.BlockSpec((BQ, 8), lambda i: (i, 0)),
            pl.BlockSpec((BQ, C), lambda i: (i, 0)),
            pl.BlockSpec((C, C), lambda i: (0, 0)),
            pl.BlockSpec((C, CP), lambda i: (0, 0)),
            pl.BlockSpec((1, C), lambda i: (0, 0)),
        ],
        out_specs=[
            pl.BlockSpec((BQ, K), lambda i: (i, 0)),
            pl.BlockSpec((BQ, C), lambda i: (i, 0)),
            pl.BlockSpec((BQ, CP), lambda i: (i, 0)),
        ],
        out_shape=[
            jax.ShapeDtypeStruct((M, K), jnp.int32),
            jax.ShapeDtypeStruct((M, C), jnp.float32),
            jax.ShapeDtypeStruct((M, CP), jnp.float32),
        ],
        scratch_shapes=[
            pltpu.VMEM((BQ, M), jnp.float32),
            pltpu.VMEM((_R, BQ, 128), jnp.float32),
            pltpu.VMEM((_R, BQ, 128), jnp.int32),
        ],
    )(pT, p8, f, wd, wg, b1)


_NC, _NS = 2, 16
_NW = _NC * _NS
_BPW = E // _NW
_CH = 512
_NCH = _BPW // _CH
_SUB = _CH // 128


def _gather_fn():
    mesh = plsc.VectorSubcoreMesh(core_axis_name="c", subcore_axis_name="s")

    @functools.partial(
        pl.kernel, mesh=mesh,
        out_type=jax.ShapeDtypeStruct((E, CP), jnp.float32),
        scratch_types=[
            pltpu.VMEM((_CH,), jnp.int32),
            pltpu.VMEM((_CH, CP), jnp.float32),
            pltpu.SemaphoreType.DMA,
        ],
    )
    def gather_k(table_hbm, idx_hbm, out_hbm, idx_v, rows_v, sem):
        wid = lax.axis_index("s") * _NC + lax.axis_index("c")
        base = wid * _BPW

        def chunk(j, carry):
            off = base + j * _CH
            pltpu.sync_copy(idx_hbm.at[pl.ds(off, _CH)], idx_v)
            cps = [
                pltpu.async_copy(
                    table_hbm.at[idx_v.at[pl.ds(t * 128, 128)]],
                    rows_v.at[pl.ds(t * 128, 128), :],
                    sem)
                for t in range(_SUB)
            ]
            for cp in cps:
                cp.wait()
            pltpu.sync_copy(rows_v, out_hbm.at[pl.ds(off, _CH)])
            return carry

        lax.fori_loop(0, _NCH, chunk, 0)

    return gather_k


def _mlp_body(g3_ref, a_ref, w2_ref, b2_ref, o_ref):
    a = a_ref[...]
    w2 = w2_ref[...]
    acc = jnp.full((BQ3, C), -jnp.inf, jnp.float32)
    for k in range(K):
        e = jnp.maximum(g3_ref[k, :, :C] + a, 0.0)
        acc = jnp.maximum(
            acc, jax.lax.dot(e, w2, preferred_element_type=jnp.float32))
    o_ref[...] = acc + b2_ref[...]


def _mlp_call(g3, a, w2t, b2):
    return pl.pallas_call(
        _mlp_body,
        grid=(M // BQ3,),
        in_specs=[
            pl.BlockSpec((K, BQ3, CP), lambda i: (0, i, 0)),
            pl.BlockSpec((BQ3, C), lambda i: (i, 0)),
            pl.BlockSpec((C, C), lambda i: (0, 0)),
            pl.BlockSpec((1, C), lambda i: (0, 0)),
        ],
        out_specs=pl.BlockSpec((BQ3, C), lambda i: (i, 0)),
        out_shape=jax.ShapeDtypeStruct((M, C), jnp.float32),
    )(g3, a, w2t, b2)


def kernel(pos, features, W1, b1, W2, b2):
    B, N, _ = pos.shape
    p = pos.reshape(M, 3)
    f = features.reshape(M, C)

    p8 = jnp.concatenate([p, jnp.zeros((M, 5), jnp.float32)], axis=1)
    pT = p8.T
    p8 = -2.0 * p8
    wd = (W1[:, :C] - W1[:, C:]).T
    wg = jnp.concatenate(
        [W1[:, C:].T, jnp.zeros((C, CP - C), jnp.float32)], axis=1)

    idx, A, G = _knn_call(pT, p8, f, wd, wg, b1.reshape(1, C))

    idxT = idx.T.reshape(E)
    gathered = _gather_fn()(G, idxT)
    g3 = gathered.reshape(K, M, CP)

    x = _mlp_call(g3, A, W2.T, b2.reshape(1, C))
    return x.reshape(B, N, C)


# speedup vs baseline: 1.0445x; 1.0445x over previous
import functools

import jax
import jax.numpy as jnp
from jax import lax
from jax.experimental import pallas as pl
from jax.experimental.pallas import tpu as pltpu
from jax.experimental.pallas import tpu_sc as plsc

M = 8192
K = 20
C = 64
CP = 128
BQ = 256
BQ3 = 512
E = M * K


_NT = M // 128
_R = 5


def _knn_body(pT_ref, p8_ref, f_ref, wd_ref, wg_ref, b1_ref,
              idx_ref, a_ref, g_ref, s_ref, wv_ref, wi_ref):
    pT = pT_ref[...]
    sq = jnp.sum(pT * pT, axis=0, keepdims=True)
    pb = p8_ref[...]
    s_ref[...] = sq + jax.lax.dot(
        pb, pT, preferred_element_type=jnp.float32)

    f = f_ref[...]
    a_ref[...] = jax.lax.dot(
        f, wd_ref[...], preferred_element_type=jnp.float32) + b1_ref[...]
    g_ref[...] = jax.lax.dot(
        f, wg_ref[...], preferred_element_type=jnp.float32)

    INF = jnp.float32(jnp.inf)
    lane128 = lax.broadcasted_iota(jnp.int32, (BQ, 128), 1)
    lane32 = lax.broadcasted_iota(jnp.int32, (BQ, 32), 1)

    def round_body(r, ti_prev):
        cm = jnp.full((BQ, 128), INF, jnp.float32)
        for t in range(_NT):
            seg = s_ref[:, t * 128:(t + 1) * 128]
            segm = jnp.where(ti_prev == t, INF, seg)
            s_ref[:, t * 128:(t + 1) * 128] = segm
            cm = jnp.minimum(cm, segm)
        arr = s_ref[...]
        ti = jnp.full((BQ, 128), _NT, jnp.int32)
        for t in range(_NT - 1, -1, -1):
            ti = jnp.where(arr[:, t * 128:(t + 1) * 128] == cm, t, ti)
        wv_ref[r] = cm
        wi_ref[r] = ti * 128 + lane128
        return ti

    lax.fori_loop(0, _R, round_body, jnp.full((BQ, 128), _NT, jnp.int32))

    BIG = jnp.int32(M)
    lane32h = lax.broadcasted_iota(jnp.int32, (128, 32), 1)
    viol = jnp.bool_(False)
    for half in range(BQ // 128):
        lo = half * 128

        def merge_body(k, carry, lo=lo):
            h, hi, ptr, acc = carry
            m = jnp.min(h, axis=1, keepdims=True)
            hit = h == m
            gm = jnp.min(jnp.where(hit, hi, BIG), axis=1, keepdims=True)
            win = hit & (hi == gm)
            ptr = ptr + win.astype(jnp.int32)
            hn = jnp.full((128, 128), INF, jnp.float32)
            hin = jnp.full((128, 128), BIG, jnp.int32)
            for r in range(_R):
                cond = ptr == r
                hn = jnp.where(cond, wv_ref[r, lo:lo + 128, :], hn)
                hin = jnp.where(cond, wi_ref[r, lo:lo + 128, :], hin)
            return hn, hin, ptr, jnp.where(lane32h == k, gm, acc)

        _, _, ptr, acc = lax.fori_loop(
            0, K, merge_body,
            (wv_ref[0, lo:lo + 128, :], wi_ref[0, lo:lo + 128, :],
             jnp.zeros((128, 128), jnp.int32), jnp.zeros((128, 32), jnp.int32)))
        idx_ref[lo:lo + 128, :] = acc[:, :K]
        viol = viol | (jnp.max(ptr) >= _R)


    def fallback():
        s_ref[...] = sq + jax.lax.dot(
            pb, pT, preferred_element_type=jnp.float32)
        iota = lax.broadcasted_iota(jnp.int32, (BQ, M), 1)

        def body(k, acc):
            arr = s_ref[...]
            m = jnp.min(arr, axis=1, keepdims=True)
            hit = arr == m
            idxv = jnp.min(jnp.where(hit, iota, M), axis=1, keepdims=True)
            s_ref[...] = jnp.where(iota == idxv, INF, arr)
            return jnp.where(lane32 == k, idxv, acc)

        acc = lax.fori_loop(0, K, body, jnp.zeros((BQ, 32), jnp.int32))
        idx_ref[...] = acc[:, :K]

    lax.cond(viol, fallback, lambda: None)


def _knn_call(pT, p8, f, wd, wg, b1):
    return pl.pallas_call(
        _knn_body,
        grid=(M // BQ,),
        in_specs=[
            pl.BlockSpec((8, M), lambda i: (0, 0)),
            pl.BlockSpec((BQ, 8), lambda i: (i, 0)),
            pl.BlockSpec((BQ, C), lambda i: (i, 0)),
            pl.BlockSpec((C, C), lambda i: (0, 0)),
            pl.BlockSpec((C, CP), lambda i: (0, 0)),
            pl.BlockSpec((1, C), lambda i: (0, 0)),
        ],
        out_specs=[
            pl.BlockSpec((BQ, K), lambda i: (i, 0)),
            pl.BlockSpec((BQ, C), lambda i: (i, 0)),
            pl.BlockSpec((BQ, CP), lambda i: (i, 0)),
        ],
        out_shape=[
            jax.ShapeDtypeStruct((M, K), jnp.int32),
            jax.ShapeDtypeStruct((M, C), jnp.float32),
            jax.ShapeDtypeStruct((M, CP), jnp.float32),
        ],
        scratch_shapes=[
            pltpu.VMEM((BQ, M), jnp.float32),
            pltpu.VMEM((_R, BQ, 128), jnp.float32),
            pltpu.VMEM((_R, BQ, 128), jnp.int32),
        ],
    )(pT, p8, f, wd, wg, b1)


_NC, _NS = 2, 16
_NW = _NC * _NS
_BPW = E // _NW
_CH = 512
_NCH = _BPW // _CH
_SUB = _CH // 128


def _gather_fn():
    mesh = plsc.VectorSubcoreMesh(core_axis_name="c", subcore_axis_name="s")

    @functools.partial(
        pl.kernel, mesh=mesh,
        out_type=jax.ShapeDtypeStruct((E, CP), jnp.float32),
        scratch_types=[
            pltpu.VMEM((_CH,), jnp.int32),
            pltpu.VMEM((_CH, CP), jnp.float32),
            pltpu.SemaphoreType.DMA,
        ],
    )
    def gather_k(table_hbm, idx_hbm, out_hbm, idx_v, rows_v, sem):
        wid = lax.axis_index("s") * _NC + lax.axis_index("c")
        base = wid * _BPW

        def chunk(j, carry):
            off = base + j * _CH
            pltpu.sync_copy(idx_hbm.at[pl.ds(off, _CH)], idx_v)
            cps = [
                pltpu.async_copy(
                    table_hbm.at[idx_v.at[pl.ds(t * 128, 128)]],
                    rows_v.at[pl.ds(t * 128, 128), :],
                    sem)
                for t in range(_SUB)
            ]
            for cp in cps:
                cp.wait()
            pltpu.sync_copy(rows_v, out_hbm.at[pl.ds(off, _CH)])
            return carry

        lax.fori_loop(0, _NCH, chunk, 0)

    return gather_k


def _mlp_body(g3_ref, a_ref, w2_ref, b2_ref, o_ref):
    a = a_ref[...]
    w2 = w2_ref[...]
    acc = jnp.full((BQ3, C), -jnp.inf, jnp.float32)
    for k in range(K):
        e = jnp.maximum(g3_ref[k, :, :C] + a, 0.0)
        acc = jnp.maximum(
            acc, jax.lax.dot(e, w2, preferred_element_type=jnp.float32))
    o_ref[...] = acc + b2_ref[...]


def _mlp_call(g3, a, w2t, b2):
    return pl.pallas_call(
        _mlp_body,
        grid=(M // BQ3,),
        in_specs=[
            pl.BlockSpec((K, BQ3, CP), lambda i: (0, i, 0)),
            pl.BlockSpec((BQ3, C), lambda i: (i, 0)),
            pl.BlockSpec((C, C), lambda i: (0, 0)),
            pl.BlockSpec((1, C), lambda i: (0, 0)),
        ],
        out_specs=pl.BlockSpec((BQ3, C), lambda i: (i, 0)),
        out_shape=jax.ShapeDtypeStruct((M, C), jnp.float32),
    )(g3, a, w2t, b2)


def kernel(pos, features, W1, b1, W2, b2):
    B, N, _ = pos.shape
    p = pos.reshape(M, 3)
    f = features.reshape(M, C)

    p8 = jnp.concatenate([p, jnp.zeros((M, 5), jnp.float32)], axis=1)
    pT = p8.T
    p8 = -2.0 * p8
    wd = (W1[:, :C] - W1[:, C:]).T
    wg = jnp.concatenate(
        [W1[:, C:].T, jnp.zeros((C, CP - C), jnp.float32)], axis=1)

    idx, A, G = _knn_call(pT, p8, f, wd, wg, b1.reshape(1, C))

    idxT = idx.T.reshape(E)
    gathered = _gather_fn()(G, idxT)
    g3 = gathered.reshape(K, M, CP)

    x = _mlp_call(g3, A, W2.T, b2.reshape(1, C))
    return x.reshape(B, N, C)


# speedup vs baseline: 1.2213x; 1.1693x over previous
import functools

import jax
import jax.numpy as jnp
from jax import lax
from jax.experimental import pallas as pl
from jax.experimental.pallas import tpu as pltpu
from jax.experimental.pallas import tpu_sc as plsc

M = 8192
K = 20
C = 64
CP = 128
BQ = 256
BQ3 = 512
E = M * K


_NT = M // 128
_R = 5


def _knn_body(pT_ref, p8_ref, f_ref, wd_ref, wg_ref, b1_ref,
              idx_ref, a_ref, g_ref, s_ref, wv_ref, wi_ref):
    pT = pT_ref[...]
    sq = jnp.sum(pT * pT, axis=0, keepdims=True)
    pb = p8_ref[...]
    s_ref[...] = sq + jax.lax.dot(
        pb, pT, preferred_element_type=jnp.float32)

    f = f_ref[...]
    a_ref[...] = jax.lax.dot(
        f, wd_ref[...], preferred_element_type=jnp.float32) + b1_ref[...]
    g_ref[...] = jax.lax.dot(
        f, wg_ref[...], preferred_element_type=jnp.float32)

    INF = jnp.float32(jnp.inf)
    lane128 = lax.broadcasted_iota(jnp.int32, (BQ, 128), 1)
    lane32 = lax.broadcasted_iota(jnp.int32, (BQ, 32), 1)

    def round_body(r, ti_prev):
        cm = jnp.full((BQ, 128), INF, jnp.float32)
        for t in range(_NT):
            seg = s_ref[:, t * 128:(t + 1) * 128]
            segm = jnp.where(ti_prev == t, INF, seg)
            s_ref[:, t * 128:(t + 1) * 128] = segm
            cm = jnp.minimum(cm, segm)
        arr = s_ref[...]
        ti = jnp.full((BQ, 128), _NT, jnp.int32)
        for t in range(_NT - 1, -1, -1):
            ti = jnp.where(arr[:, t * 128:(t + 1) * 128] == cm, t, ti)
        wv_ref[r] = cm
        wi_ref[r] = ti * 128 + lane128
        return ti

    lax.fori_loop(0, _R, round_body, jnp.full((BQ, 128), _NT, jnp.int32))

    BIG = jnp.int32(M)

    def merge_body(k, carry):
        h, hi, ptr, acc = carry
        m = jnp.min(h, axis=1, keepdims=True)
        hit = h == m
        gm = jnp.min(jnp.where(hit, hi, BIG), axis=1, keepdims=True)
        win = hit & (hi == gm)
        ptr = ptr + win.astype(jnp.int32)
        hn = jnp.full((BQ, 128), INF, jnp.float32)
        hin = jnp.full((BQ, 128), BIG, jnp.int32)
        for r in range(_R):
            cond = ptr == r
            hn = jnp.where(cond, wv_ref[r], hn)
            hin = jnp.where(cond, wi_ref[r], hin)
        return hn, hin, ptr, jnp.where(lane32 == k, gm, acc)

    _, _, ptr, acc = lax.fori_loop(
        0, K, merge_body,
        (wv_ref[0], wi_ref[0], jnp.zeros((BQ, 128), jnp.int32),
         jnp.zeros((BQ, 32), jnp.int32)))
    idx_ref[...] = acc[:, :K]

    viol = jnp.max(ptr) >= _R

    def fallback():
        s_ref[...] = sq + jax.lax.dot(
            pb, pT, preferred_element_type=jnp.float32)
        iota = lax.broadcasted_iota(jnp.int32, (BQ, M), 1)

        def body(k, acc):
            arr = s_ref[...]
            m = jnp.min(arr, axis=1, keepdims=True)
            hit = arr == m
            idxv = jnp.min(jnp.where(hit, iota, M), axis=1, keepdims=True)
            s_ref[...] = jnp.where(iota == idxv, INF, arr)
            return jnp.where(lane32 == k, idxv, acc)

        acc = lax.fori_loop(0, K, body, jnp.zeros((BQ, 32), jnp.int32))
        idx_ref[...] = acc[:, :K]

    lax.cond(viol, fallback, lambda: None)


def _knn_call(pT, p8, f, wd, wg, b1):
    return pl.pallas_call(
        _knn_body,
        grid=(M // BQ,),
        in_specs=[
            pl.BlockSpec((8, M), lambda i: (0, 0)),
            pl.BlockSpec((BQ, 8), lambda i: (i, 0)),
            pl.BlockSpec((BQ, C), lambda i: (i, 0)),
            pl.BlockSpec((C, C), lambda i: (0, 0)),
            pl.BlockSpec((C, CP), lambda i: (0, 0)),
            pl.BlockSpec((1, C), lambda i: (0, 0)),
        ],
        out_specs=[
            pl.BlockSpec((BQ, K), lambda i: (i, 0)),
            pl.BlockSpec((BQ, C), lambda i: (i, 0)),
            pl.BlockSpec((BQ, CP), lambda i: (i, 0)),
        ],
        out_shape=[
            jax.ShapeDtypeStruct((M, K), jnp.int32),
            jax.ShapeDtypeStruct((M, C), jnp.float32),
            jax.ShapeDtypeStruct((M, CP), jnp.float32),
        ],
        scratch_shapes=[
            pltpu.VMEM((BQ, M), jnp.float32),
            pltpu.VMEM((_R, BQ, 128), jnp.float32),
            pltpu.VMEM((_R, BQ, 128), jnp.int32),
        ],
    )(pT, p8, f, wd, wg, b1)


_NC, _NS = 2, 16
_NW = _NC * _NS
_BPW = E // _NW
_CH = 512
_NCH = _BPW // _CH
_SUB = _CH // 128


def _gather_fn():
    mesh = plsc.VectorSubcoreMesh(core_axis_name="c", subcore_axis_name="s")

    @functools.partial(
        pl.kernel, mesh=mesh,
        out_type=jax.ShapeDtypeStruct((E, CP), jnp.float32),
        scratch_types=[
            pltpu.VMEM((_CH,), jnp.int32),
            pltpu.VMEM((_CH, CP), jnp.float32),
            pltpu.SemaphoreType.DMA,
        ],
    )
    def gather_k(table_hbm, idx_hbm, out_hbm, idx_v, rows_v, sem):
        wid = lax.axis_index("s") * _NC + lax.axis_index("c")
        base = wid * _BPW

        def chunk(j, carry):
            off = base + j * _CH
            pltpu.sync_copy(idx_hbm.at[pl.ds(off, _CH)], idx_v)
            cps = [
                pltpu.async_copy(
                    table_hbm.at[idx_v.at[pl.ds(t * 128, 128)]],
                    rows_v.at[pl.ds(t * 128, 128), :],
                    sem)
                for t in range(_SUB)
            ]
            for cp in cps:
                cp.wait()
            pltpu.sync_copy(rows_v, out_hbm.at[pl.ds(off, _CH)])
            return carry

        lax.fori_loop(0, _NCH, chunk, 0)

    return gather_k


def _mlp_body(g3_ref, a_ref, w2_ref, b2_ref, o_ref):
    a = a_ref[...]
    w2 = w2_ref[...]
    acc = jnp.full((BQ3, C), -jnp.inf, jnp.float32)
    for k in range(K):
        e = jnp.maximum(g3_ref[k, :, :C] + a, 0.0)
        acc = jnp.maximum(
            acc, jax.lax.dot(e, w2, preferred_element_type=jnp.float32))
    o_ref[...] = acc + b2_ref[...]


def _mlp_call(g3, a, w2t, b2):
    return pl.pallas_call(
        _mlp_body,
        grid=(M // BQ3,),
        in_specs=[
            pl.BlockSpec((K, BQ3, CP), lambda i: (0, i, 0)),
            pl.BlockSpec((BQ3, C), lambda i: (i, 0)),
            pl.BlockSpec((C, C), lambda i: (0, 0)),
            pl.BlockSpec((1, C), lambda i: (0, 0)),
        ],
        out_specs=pl.BlockSpec((BQ3, C), lambda i: (i, 0)),
        out_shape=jax.ShapeDtypeStruct((M, C), jnp.float32),
    )(g3, a, w2t, b2)


def kernel(pos, features, W1, b1, W2, b2):
    B, N, _ = pos.shape
    p = pos.reshape(M, 3)
    f = features.reshape(M, C)

    p8 = jnp.concatenate([p, jnp.zeros((M, 5), jnp.float32)], axis=1)
    pT = p8.T
    p8 = -2.0 * p8
    wd = (W1[:, :C] - W1[:, C:]).T
    wg = jnp.concatenate(
        [W1[:, C:].T, jnp.zeros((C, CP - C), jnp.float32)], axis=1)

    idx, A, G = _knn_call(pT, p8, f, wd, wg, b1.reshape(1, C))

    idxT = idx.T.reshape(E)
    gathered = _gather_fn()(G, idxT)
    g3 = gathered.reshape(K, M, CP)

    x = _mlp_call(g3, A, W2.T, b2.reshape(1, C))
    return x.reshape(B, N, C)
